# baseline (speedup 1.0000x reference)
import functools

import numpy as np

import jax
import jax.numpy as jnp
from jax import lax
from jax.experimental import pallas as pl
from jax.experimental.pallas import tpu as pltpu
from jax.experimental.pallas import tpu_sc as plsc

NC = 2
NS = 16
NW = NC * NS
L = 16

KCHUNK = 80


def _mm3_body(x_ref, w_ref, o_ref):
    o_ref[0] = jnp.dot(x_ref[...], w_ref[0],
                       preferred_element_type=jnp.float32)


def _tc_matmul3(x, w3, bn):
    n, feat = x.shape
    nb, _, hid = w3.shape
    grid = (nb, n // bn)
    return pl.pallas_call(
        _mm3_body,
        grid=grid,
        in_specs=[
            pl.BlockSpec((bn, feat), lambda b, i: (i, 0)),
            pl.BlockSpec((1, feat, hid), lambda b, i: (b, 0, 0)),
        ],
        out_specs=pl.BlockSpec((1, bn, hid), lambda b, i: (b, i, 0)),
        out_shape=jax.ShapeDtypeStruct((nb, n, hid), jnp.float32),
    )(x, w3)


def _layer2_body(p_ref, w_ref, o_ref):
    h = jax.nn.relu(p_ref[0, 0] + p_ref[0, 1])
    o_ref[0] = jnp.dot(h, w_ref[0], preferred_element_type=jnp.float32)


def _tc_layer2(parts, w3, bn):
    nb, two, n, hid = parts.shape
    _, _, cls = w3.shape
    grid = (nb, n // bn)
    return pl.pallas_call(
        _layer2_body,
        grid=grid,
        in_specs=[
            pl.BlockSpec((1, two, bn, hid), lambda b, i: (b, 0, i, 0)),
            pl.BlockSpec((1, hid, cls), lambda b, i: (b, 0, 0)),
        ],
        out_specs=pl.BlockSpec((1, bn, cls), lambda b, i: (b, i, 0)),
        out_shape=jax.ShapeDtypeStruct((nb, n, cls), jnp.float32),
    )(parts, w3)


def _addrelu_body(p_ref, o_ref):
    o_ref[0] = jax.nn.relu(p_ref[0, 0] + p_ref[0, 1])


def _tc_addrelu(parts, bn):
    nb, two, n, hid = parts.shape
    grid = (nb, n // bn)
    return pl.pallas_call(
        _addrelu_body,
        grid=grid,
        in_specs=[
            pl.BlockSpec((1, two, bn, hid), lambda b, i: (b, 0, i, 0)),
        ],
        out_specs=pl.BlockSpec((1, bn, hid), lambda b, i: (b, i, 0)),
        out_shape=jax.ShapeDtypeStruct((nb, n, hid), jnp.float32),
    )(parts)


def _final_mm_body(p_ref, w2_ref, bc_ref, g_ref, o_ref):
    g = g_ref[0, 0]
    x1 = jnp.dot(p_ref[0, 0] + p_ref[0, 1], w2_ref[0],
                 preferred_element_type=jnp.float32)
    x2 = jnp.dot(p_ref[1, 0] + p_ref[1, 1], w2_ref[1],
                 preferred_element_type=jnp.float32)
    xh = jnp.dot(p_ref[2, 0] + p_ref[2, 1], w2_ref[2],
                 preferred_element_type=jnp.float32)
    xf = g * x1 + (1.0 - g) * (x2 + xh) + bc_ref[0][None, :]
    m = jnp.max(xf, axis=-1, keepdims=True)
    s = xf - m
    lse = jnp.log(jnp.sum(jnp.exp(s), axis=-1, keepdims=True))
    o_ref[...] = s - lse


def _tc_final_mm(parts, w2, bias_comb, gate, n_out, bn):
    nb, two, _, hid = parts.shape
    cls = w2.shape[2]
    grid = (n_out // bn,)
    return pl.pallas_call(
        _final_mm_body,
        grid=grid,
        in_specs=[
            pl.BlockSpec((nb, two, bn, hid), lambda i: (0, 0, i, 0)),
            pl.BlockSpec((nb, hid, cls), lambda i: (0, 0, 0)),
            pl.BlockSpec((8, cls), lambda i: (0, 0)),
            pl.BlockSpec((8, 128), lambda i: (0, 0)),
        ],
        out_specs=pl.BlockSpec((bn, cls), lambda i: (i, 0)),
        out_shape=jax.ShapeDtypeStruct((n_out, cls), jnp.float32),
    )(parts, w2, bias_comb, gate)


def _final_body(p_ref, bc_ref, g_ref, o_ref):
    g = g_ref[0, 0]
    x1 = p_ref[0, 0] + p_ref[0, 1]
    x2 = p_ref[1, 0] + p_ref[1, 1]
    xh = p_ref[2, 0] + p_ref[2, 1]
    xf = g * x1 + (1.0 - g) * (x2 + xh) + bc_ref[0][None, :]
    m = jnp.max(xf, axis=-1, keepdims=True)
    s = xf - m
    lse = jnp.log(jnp.sum(jnp.exp(s), axis=-1, keepdims=True))
    o_ref[...] = s - lse


def _tc_final(parts, bias_comb, gate, n_out, bn):
    nb, two, _, cls = parts.shape
    grid = (n_out // bn,)
    return pl.pallas_call(
        _final_body,
        grid=grid,
        in_specs=[
            pl.BlockSpec((nb, two, bn, cls), lambda i: (0, 0, i, 0)),
            pl.BlockSpec((8, cls), lambda i: (0, 0)),
            pl.BlockSpec((8, 128), lambda i: (0, 0)),
        ],
        out_specs=pl.BlockSpec((bn, cls), lambda i: (i, 0)),
        out_shape=jax.ShapeDtypeStruct((n_out, cls), jnp.float32),
    )(parts, bias_comb, gate)


_DNUMS = lax.GatherDimensionNumbers(
    offset_dims=(), collapsed_slice_dims=(0,), start_index_map=(0,))


def _make_spmm(sup_rows, n_edges, feat, nbranch, npad, tc_tiling=True):
    ept = n_edges // NW
    sb_sz = KCHUNK
    nsb = ept // sb_sz
    assert ept % sb_sz == 0 and nsb % 2 == 1 and sb_sz % L == 0
    rows_per_tile = npad // NS
    zc = KCHUNK
    nzc = rows_per_tile // zc
    assert rows_per_tile % zc == 0
    nf = feat // L

    mesh = plsc.VectorSubcoreMesh(core_axis_name="c", subcore_axis_name="s")

    @functools.partial(
        pl.kernel,
        out_type=jax.ShapeDtypeStruct((nbranch * NC * npad, feat),
                                      jnp.float32),
        mesh=mesh,
        scratch_types=[
            pltpu.VMEM_SHARED((npad, feat), jnp.float32),
            pltpu.VMEM((ept,), jnp.int32),
            pltpu.VMEM((ept,), jnp.float32),
            pltpu.VMEM((sb_sz, feat), jnp.float32),
            pltpu.VMEM((sb_sz, feat), jnp.float32),
            [pltpu.VMEM((KCHUNK,), jnp.int32) for _ in range(2)],
            pltpu.SemaphoreType.DMA,
            pltpu.SemaphoreType.DMA,
            pltpu.SemaphoreType.DMA,
            pltpu.SemaphoreType.DMA,
        ],
        compiler_params=(None if tc_tiling else pltpu.CompilerParams(
            use_tc_tiling_on_sc=False)),
    )
    def spmm(sup_hbm, src_hbm, dst_hbm, w_hbm, out_hbm,
             acc, src_big, w_big, rows_a, rows_b, dst_refs,
             gsem_a, gsem_b, dsem_a, dsem_b):
        c = lax.axis_index("c")
        s = lax.axis_index("s")
        wid = c * NS + s
        base_e = wid * ept
        row0 = s * rows_per_tile
        bufs = [
            dict(rows=rows_a, dst=dst_refs[0], gsem=gsem_a, dsem=dsem_a),
            dict(rows=rows_b, dst=dst_refs[1], gsem=gsem_b, dsem=dsem_b),
        ]

        def prep(gbase, sb, buf):
            e0 = sb * sb_sz
            pltpu.async_copy(dst_hbm.at[pl.ds(gbase + e0, KCHUNK)],
                             buf["dst"], buf["dsem"])
            pltpu.async_copy(sup_hbm.at[src_big.at[pl.ds(e0, KCHUNK)]],
                             buf["rows"], buf["gsem"])

        def process(sb, buf):
            e0 = sb * sb_sz
            rows = buf["rows"]
            pltpu.make_async_copy(sup_hbm.at[pl.ds(0, sb_sz)], rows,
                                  buf["gsem"]).wait()

            @pl.loop(0, sb_sz // L)
            def _(g):
                wv16 = w_big[pl.ds(e0 + g * L, L)]
                for j2 in range(L):
                    idx = (jnp.zeros((L,), jnp.int32) + j2)[:, None]
                    splat = lax.gather(
                        wv16, idx, _DNUMS, slice_sizes=(1,),
                        mode=lax.GatherScatterMode.PROMISE_IN_BOUNDS)
                    row = g * L + j2
                    for f in range(nf):
                        sl = pl.ds(f * L, L)
                        rows[row, sl] = rows[row, sl] * splat

            pltpu.make_async_copy(dst_hbm.at[pl.ds(0, KCHUNK)],
                                  buf["dst"], buf["dsem"]).wait()
            pltpu.sync_copy(rows, acc.at[buf["dst"]], add=True)

        def zero_tmp():
            @pl.loop(0, zc)
            def _(j):
                for f in range(nf):
                    rows_a[j, pl.ds(f * L, L)] = jnp.zeros((L,), jnp.float32)

        def zero_my_slice():
            for k in range(nzc):
                pltpu.sync_copy(rows_a, acc.at[pl.ds(row0 + k * zc, zc)])

        zero_tmp()
        zero_my_slice()

        @pl.loop(0, nbranch)
        def _(b):
            gbase = b * n_edges + base_e
            pltpu.sync_copy(src_hbm.at[pl.ds(gbase, ept)], src_big)
            pltpu.sync_copy(w_hbm.at[pl.ds(gbase, ept)], w_big)
            boff = jnp.zeros((L,), jnp.int32) + b * sup_rows

            @pl.loop(0, ept // L)
            def _(i):
                sl = pl.ds(i * L, L)
                src_big[sl] = src_big[sl] + boff

            plsc.subcore_barrier()
            prep(gbase, 0, bufs[0])

            @pl.loop(0, (nsb - 1) // 2)
            def _(p):
                prep(gbase, 2 * p + 1, bufs[1])
                process(2 * p, bufs[0])
                prep(gbase, 2 * p + 2, bufs[0])
                process(2 * p + 1, bufs[1])

            process(nsb - 1, bufs[0])
            plsc.subcore_barrier()
            part = b * NC + c
            for k in range(nzc):
                r = row0 + k * zc
                pltpu.sync_copy(acc.at[pl.ds(r, zc)], rows_a)
                pltpu.sync_copy(rows_a, out_hbm.at[pl.ds(part * npad + r,
                                                         zc)])
            zero_tmp()
            zero_my_slice()

    return spmm


def kernel(x, ei_mid, ew_mid, ei_low, ew_low, ei_high, ew_high,
           W_h0, W_out, b_out, W_aH, W_aO, b_aO, W_aHH, W_aOH, b_aOH, gate2):
    n, feat = x.shape
    hid = W_h0.shape[1]
    cls = W_out.shape[1]
    e = ew_mid.shape[0]
    bn = 400

    src_all = jnp.stack([ei_mid[0], ei_low[0], ei_high[0]]).astype(jnp.int32)
    dst_all = jnp.stack([ei_mid[1], ei_low[1], ei_high[1]]).astype(jnp.int32)
    w_all = jnp.stack([ew_mid, ew_low, ew_high])
    w1 = jnp.stack([W_h0, W_aH, W_aHH])
    w2 = jnp.stack([W_out, W_aO, W_aOH])

    g = gate2[0]
    bias_comb = g * b_out + (1.0 - g) * (b_aO + b_aOH)
    bias_comb = jnp.broadcast_to(bias_comb[None, :], (8, cls))
    gate_b = jnp.broadcast_to(gate2[:, None], (8, 128))

    npad = -(-n // (NS * 128)) * (NS * 128)

    sup3 = _tc_matmul3(x, w1, bn)
    spmm1 = _make_spmm(n, e, hid, 3, npad)
    p1 = spmm1(sup3.reshape(3 * n, hid), src_all.reshape(-1),
               dst_all.reshape(-1), w_all.reshape(-1))
    p1 = p1.reshape(3, NC, npad, hid)

    h = _tc_layer2(p1, w2, npad // 16)
    spmm2 = _make_spmm(npad, e, cls, 3, npad, tc_tiling=False)
    p2 = spmm2(h.reshape(3 * npad, cls), src_all.reshape(-1),
               dst_all.reshape(-1), w_all.reshape(-1))
    p2 = p2.reshape(3, NC, npad, cls)

    return _tc_final(p2, bias_comb, gate_b, n, bn)


# speedup vs baseline: 1.2224x; 1.2224x over previous
import functools

import numpy as np

import jax
import jax.numpy as jnp
from jax import lax
from jax.experimental import pallas as pl
from jax.experimental.pallas import tpu as pltpu
from jax.experimental.pallas import tpu_sc as plsc

NC = 2
NS = 16
NW = NC * NS
L = 16

KCHUNK = 80


def _mm3_body(x_ref, w_ref, o_ref):
    o_ref[0] = jnp.dot(x_ref[...], w_ref[0],
                       preferred_element_type=jnp.float32)


def _tc_matmul3(x, w3, bn):
    n, feat = x.shape
    nb, _, hid = w3.shape
    grid = (nb, n // bn)
    return pl.pallas_call(
        _mm3_body,
        grid=grid,
        in_specs=[
            pl.BlockSpec((bn, feat), lambda b, i: (i, 0)),
            pl.BlockSpec((1, feat, hid), lambda b, i: (b, 0, 0)),
        ],
        out_specs=pl.BlockSpec((1, bn, hid), lambda b, i: (b, i, 0)),
        out_shape=jax.ShapeDtypeStruct((nb, n, hid), jnp.float32),
    )(x, w3)


def _layer2_body(p_ref, w_ref, o_ref):
    h = jax.nn.relu(p_ref[0, 0] + p_ref[0, 1])
    o_ref[0] = jnp.dot(h, w_ref[0], preferred_element_type=jnp.float32)


def _tc_layer2(parts, w3, bn):
    nb, two, n, hid = parts.shape
    _, _, cls = w3.shape
    grid = (nb, n // bn)
    return pl.pallas_call(
        _layer2_body,
        grid=grid,
        in_specs=[
            pl.BlockSpec((1, two, bn, hid), lambda b, i: (b, 0, i, 0)),
            pl.BlockSpec((1, hid, cls), lambda b, i: (b, 0, 0)),
        ],
        out_specs=pl.BlockSpec((1, bn, cls), lambda b, i: (b, i, 0)),
        out_shape=jax.ShapeDtypeStruct((nb, n, cls), jnp.float32),
    )(parts, w3)


def _addrelu_body(p_ref, o_ref):
    o_ref[0] = jax.nn.relu(p_ref[0, 0] + p_ref[0, 1])


def _tc_addrelu(parts, bn):
    nb, two, n, hid = parts.shape
    grid = (nb, n // bn)
    return pl.pallas_call(
        _addrelu_body,
        grid=grid,
        in_specs=[
            pl.BlockSpec((1, two, bn, hid), lambda b, i: (b, 0, i, 0)),
        ],
        out_specs=pl.BlockSpec((1, bn, hid), lambda b, i: (b, i, 0)),
        out_shape=jax.ShapeDtypeStruct((nb, n, hid), jnp.float32),
    )(parts)


def _final_mm_body(p_ref, w2_ref, bc_ref, g_ref, o_ref):
    g = g_ref[0, 0]
    x1 = jnp.dot(p_ref[0, 0] + p_ref[0, 1], w2_ref[0],
                 preferred_element_type=jnp.float32)
    x2 = jnp.dot(p_ref[1, 0] + p_ref[1, 1], w2_ref[1],
                 preferred_element_type=jnp.float32)
    xh = jnp.dot(p_ref[2, 0] + p_ref[2, 1], w2_ref[2],
                 preferred_element_type=jnp.float32)
    xf = g * x1 + (1.0 - g) * (x2 + xh) + bc_ref[0][None, :]
    m = jnp.max(xf, axis=-1, keepdims=True)
    s = xf - m
    lse = jnp.log(jnp.sum(jnp.exp(s), axis=-1, keepdims=True))
    o_ref[...] = s - lse


def _tc_final_mm(parts, w2, bias_comb, gate, n_out, bn):
    nb, two, _, hid = parts.shape
    cls = w2.shape[2]
    grid = (n_out // bn,)
    return pl.pallas_call(
        _final_mm_body,
        grid=grid,
        in_specs=[
            pl.BlockSpec((nb, two, bn, hid), lambda i: (0, 0, i, 0)),
            pl.BlockSpec((nb, hid, cls), lambda i: (0, 0, 0)),
            pl.BlockSpec((8, cls), lambda i: (0, 0)),
            pl.BlockSpec((8, 128), lambda i: (0, 0)),
        ],
        out_specs=pl.BlockSpec((bn, cls), lambda i: (i, 0)),
        out_shape=jax.ShapeDtypeStruct((n_out, cls), jnp.float32),
    )(parts, w2, bias_comb, gate)


def _final_body(p_ref, bc_ref, g_ref, o_ref):
    g = g_ref[0, 0]
    x1 = p_ref[0, 0] + p_ref[0, 1]
    x2 = p_ref[1, 0] + p_ref[1, 1]
    xh = p_ref[2, 0] + p_ref[2, 1]
    xf = g * x1 + (1.0 - g) * (x2 + xh) + bc_ref[0][None, :]
    m = jnp.max(xf, axis=-1, keepdims=True)
    s = xf - m
    lse = jnp.log(jnp.sum(jnp.exp(s), axis=-1, keepdims=True))
    o_ref[...] = s - lse


def _tc_final(parts, bias_comb, gate, n_out, bn):
    nb, two, _, cls = parts.shape
    grid = (n_out // bn,)
    return pl.pallas_call(
        _final_body,
        grid=grid,
        in_specs=[
            pl.BlockSpec((nb, two, bn, cls), lambda i: (0, 0, i, 0)),
            pl.BlockSpec((8, cls), lambda i: (0, 0)),
            pl.BlockSpec((8, 128), lambda i: (0, 0)),
        ],
        out_specs=pl.BlockSpec((bn, cls), lambda i: (i, 0)),
        out_shape=jax.ShapeDtypeStruct((n_out, cls), jnp.float32),
    )(parts, bias_comb, gate)


_DNUMS = lax.GatherDimensionNumbers(
    offset_dims=(), collapsed_slice_dims=(0,), start_index_map=(0,))


def _make_spmm(sup_rows, n_edges, feat, nbranch, npad, tc_tiling=True):
    ept = n_edges // NW
    sb_sz = KCHUNK
    nsb = ept // sb_sz
    assert ept % sb_sz == 0 and nsb % 2 == 1 and sb_sz % L == 0
    rows_per_tile = npad // NS
    zc = KCHUNK
    nzc = rows_per_tile // zc
    assert rows_per_tile % zc == 0
    nf = feat // L

    mesh = plsc.VectorSubcoreMesh(core_axis_name="c", subcore_axis_name="s")

    @functools.partial(
        pl.kernel,
        out_type=jax.ShapeDtypeStruct((nbranch * NC * npad, feat),
                                      jnp.float32),
        mesh=mesh,
        scratch_types=[
            pltpu.VMEM_SHARED((npad, feat), jnp.float32),
            pltpu.VMEM((ept,), jnp.int32),
            pltpu.VMEM((ept,), jnp.float32),
            pltpu.VMEM((sb_sz, feat), jnp.float32),
            pltpu.VMEM((sb_sz, feat), jnp.float32),
            [pltpu.VMEM((KCHUNK,), jnp.int32) for _ in range(2)],
            pltpu.SemaphoreType.DMA,
            pltpu.SemaphoreType.DMA,
            pltpu.SemaphoreType.DMA,
            pltpu.SemaphoreType.DMA,
        ],
        compiler_params=(None if tc_tiling else pltpu.CompilerParams(
            use_tc_tiling_on_sc=False)),
    )
    def spmm(sup_hbm, src_hbm, dst_hbm, w_hbm, out_hbm,
             acc, src_big, w_big, rows_a, rows_b, dst_refs,
             gsem_a, gsem_b, dsem_a, dsem_b):
        c = lax.axis_index("c")
        s = lax.axis_index("s")
        wid = c * NS + s
        base_e = wid * ept
        row0 = s * rows_per_tile
        bufs = [
            dict(rows=rows_a, dst=dst_refs[0], gsem=gsem_a, dsem=dsem_a),
            dict(rows=rows_b, dst=dst_refs[1], gsem=gsem_b, dsem=dsem_b),
        ]

        def prep(gbase, sb, buf):
            e0 = sb * sb_sz
            pltpu.async_copy(dst_hbm.at[pl.ds(gbase + e0, KCHUNK)],
                             buf["dst"], buf["dsem"])
            half = KCHUNK // 2
            pltpu.async_copy(sup_hbm.at[src_big.at[pl.ds(e0, half)]],
                             buf["rows"].at[pl.ds(0, half)], buf["gsem"])
            pltpu.async_copy(sup_hbm.at[src_big.at[pl.ds(e0 + half, half)]],
                             buf["rows"].at[pl.ds(half, half)], buf["gsem"])

        def process(sb, buf):
            e0 = sb * sb_sz
            rows = buf["rows"]
            pltpu.make_async_copy(sup_hbm.at[pl.ds(0, sb_sz)], rows,
                                  buf["gsem"]).wait()

            @pl.loop(0, sb_sz // L)
            def _(g):
                wv16 = w_big[pl.ds(e0 + g * L, L)]
                for j2 in range(L):
                    idx = (jnp.zeros((L,), jnp.int32) + j2)[:, None]
                    splat = lax.gather(
                        wv16, idx, _DNUMS, slice_sizes=(1,),
                        mode=lax.GatherScatterMode.PROMISE_IN_BOUNDS)
                    row = g * L + j2
                    for f in range(nf):
                        sl = pl.ds(f * L, L)
                        rows[row, sl] = rows[row, sl] * splat

            pltpu.make_async_copy(dst_hbm.at[pl.ds(0, KCHUNK)],
                                  buf["dst"], buf["dsem"]).wait()
            pltpu.sync_copy(rows, acc.at[buf["dst"]], add=True)

        def zero_tmp():
            @pl.loop(0, zc)
            def _(j):
                for f in range(nf):
                    rows_a[j, pl.ds(f * L, L)] = jnp.zeros((L,), jnp.float32)

        def zero_my_slice():
            for k in range(nzc):
                pltpu.sync_copy(rows_a, acc.at[pl.ds(row0 + k * zc, zc)])

        zero_tmp()
        zero_my_slice()

        @pl.loop(0, nbranch)
        def _(b):
            gbase = b * n_edges + base_e
            pltpu.sync_copy(src_hbm.at[pl.ds(gbase, ept)], src_big)
            pltpu.sync_copy(w_hbm.at[pl.ds(gbase, ept)], w_big)
            boff = jnp.zeros((L,), jnp.int32) + b * sup_rows

            @pl.loop(0, ept // L)
            def _(i):
                sl = pl.ds(i * L, L)
                src_big[sl] = src_big[sl] + boff

            plsc.subcore_barrier()
            prep(gbase, 0, bufs[0])

            @pl.loop(0, (nsb - 1) // 2)
            def _(p):
                prep(gbase, 2 * p + 1, bufs[1])
                process(2 * p, bufs[0])
                prep(gbase, 2 * p + 2, bufs[0])
                process(2 * p + 1, bufs[1])

            process(nsb - 1, bufs[0])
            plsc.subcore_barrier()
            part = b * NC + c
            for k in range(nzc):
                r = row0 + k * zc
                pltpu.sync_copy(acc.at[pl.ds(r, zc)], rows_a)
                pltpu.sync_copy(rows_a, out_hbm.at[pl.ds(part * npad + r,
                                                         zc)])
            zero_tmp()
            zero_my_slice()

    return spmm


def kernel(x, ei_mid, ew_mid, ei_low, ew_low, ei_high, ew_high,
           W_h0, W_out, b_out, W_aH, W_aO, b_aO, W_aHH, W_aOH, b_aOH, gate2):
    n, feat = x.shape
    hid = W_h0.shape[1]
    cls = W_out.shape[1]
    e = ew_mid.shape[0]
    bn = 400

    src_all = jnp.stack([ei_mid[0], ei_low[0], ei_high[0]]).astype(jnp.int32)
    dst_all = jnp.stack([ei_mid[1], ei_low[1], ei_high[1]]).astype(jnp.int32)
    w_all = jnp.stack([ew_mid, ew_low, ew_high])
    w1 = jnp.stack([W_h0, W_aH, W_aHH])
    w2 = jnp.stack([W_out, W_aO, W_aOH])

    g = gate2[0]
    bias_comb = g * b_out + (1.0 - g) * (b_aO + b_aOH)
    bias_comb = jnp.broadcast_to(bias_comb[None, :], (8, cls))
    gate_b = jnp.broadcast_to(gate2[:, None], (8, 128))

    npad = -(-n // (NS * 128)) * (NS * 128)

    sup3 = _tc_matmul3(x, w1, bn)
    spmm1 = _make_spmm(n, e, hid, 3, npad)
    p1 = spmm1(sup3.reshape(3 * n, hid), src_all.reshape(-1),
               dst_all.reshape(-1), w_all.reshape(-1))
    p1 = p1.reshape(3, NC, npad, hid)

    h = _tc_addrelu(p1, npad // 16)
    spmm2 = _make_spmm(npad, e, hid, 3, npad)
    p2 = spmm2(h.reshape(3 * npad, hid), src_all.reshape(-1),
               dst_all.reshape(-1), w_all.reshape(-1))
    p2 = p2.reshape(3, NC, npad, hid)

    return _tc_final_mm(p2, w2, bias_comb, gate_b, n, bn)
